# Initial kernel scaffold; baseline (speedup 1.0000x reference)
import functools

import jax
import jax.numpy as jnp
from jax import lax
from jax.experimental import pallas as pl
from jax.experimental.pallas import tpu as pltpu
from jax.experimental.pallas import tpu_sc as plsc

EMBEDDING_DIM = 32
_NC = 2
_NS = 16
_NW = _NC * _NS


@functools.lru_cache(maxsize=None)
def _make_gather(total_rows: int, dim: int, chunk: int):
    b_per_w = total_rows // _NW
    nchunks = b_per_w // chunk
    mesh = plsc.VectorSubcoreMesh(core_axis_name="c", subcore_axis_name="s")

    @functools.partial(
        pl.kernel,
        mesh=mesh,
        out_type=jax.ShapeDtypeStruct((total_rows, dim), jnp.float32),
        scratch_types=[
            pltpu.VMEM((chunk,), jnp.int32),
            pltpu.VMEM((chunk, dim), jnp.float32),
            pltpu.SemaphoreType.DMA,
        ],
    )
    def gather_kernel(idx_hbm, table_hbm, out_hbm, idx_v, rows_v, sem):
        wid = lax.axis_index("s") * _NC + lax.axis_index("c")
        base = wid * b_per_w

        def body(i, carry):
            off = base + i * chunk
            pltpu.sync_copy(idx_hbm.at[pl.ds(off, chunk)], idx_v)
            pltpu.async_copy(table_hbm.at[idx_v], rows_v, sem).wait()
            pltpu.sync_copy(rows_v, out_hbm.at[pl.ds(off, chunk)])
            return carry

        lax.fori_loop(0, nchunks, body, 0)

    return gather_kernel


def kernel(token_ids, weights):
    b, h = token_ids.shape
    idx = token_ids.reshape(-1).astype(jnp.int32)
    out = _make_gather(b * h, EMBEDDING_DIM, 1600)(idx, weights)
    return out.reshape(b, h, EMBEDDING_DIM)


# speedup vs baseline: 1.1036x; 1.1036x over previous
import functools

import jax
import jax.numpy as jnp
from jax import lax
from jax.experimental import pallas as pl
from jax.experimental.pallas import tpu as pltpu
from jax.experimental.pallas import tpu_sc as plsc

EMBEDDING_DIM = 32
_NC = 2
_NS = 16
_NW = _NC * _NS


@functools.lru_cache(maxsize=None)
def _make_gather(total_rows: int, dim: int, chunk: int):
    b_per_w = total_rows // _NW
    nchunks = b_per_w // chunk
    mesh = plsc.VectorSubcoreMesh(core_axis_name="c", subcore_axis_name="s")

    @functools.partial(
        pl.kernel,
        mesh=mesh,
        out_type=jax.ShapeDtypeStruct((total_rows, dim), jnp.float32),
        scratch_types=[
            pltpu.VMEM((chunk,), jnp.int32),
            pltpu.VMEM((chunk, dim), jnp.float32),
            pltpu.SemaphoreType.DMA,
        ],
        compiler_params=pltpu.CompilerParams(use_tc_tiling_on_sc=False),
    )
    def gather_kernel(idx_hbm, table_hbm, out_hbm, idx_v, rows_v, sem):
        wid = lax.axis_index("s") * _NC + lax.axis_index("c")
        base = wid * b_per_w

        def body(i, carry):
            off = base + i * chunk
            pltpu.sync_copy(idx_hbm.at[pl.ds(off, chunk)], idx_v)
            pltpu.async_copy(table_hbm.at[idx_v], rows_v, sem).wait()
            pltpu.sync_copy(rows_v, out_hbm.at[pl.ds(off, chunk)])
            return carry

        lax.fori_loop(0, nchunks, body, 0)

    return gather_kernel


def kernel(token_ids, weights):
    b, h = token_ids.shape
    idx = token_ids.reshape(-1).astype(jnp.int32)
    out = _make_gather(b * h, EMBEDDING_DIM, 1600)(idx, weights)
    return out.reshape(b, h, EMBEDDING_DIM)


# speedup vs baseline: 1.1110x; 1.0067x over previous
import functools

import jax
import jax.numpy as jnp
from jax import lax
from jax.experimental import pallas as pl
from jax.experimental.pallas import tpu as pltpu
from jax.experimental.pallas import tpu_sc as plsc

EMBEDDING_DIM = 32
_NC = 2
_NS = 16
_NW = _NC * _NS
_NBUF = 2


@functools.lru_cache(maxsize=None)
def _make_gather(total_rows: int, dim: int, chunk: int):
    b_per_w = total_rows // _NW
    nchunks = b_per_w // chunk
    assert nchunks % _NBUF == 0
    mesh = plsc.VectorSubcoreMesh(core_axis_name="c", subcore_axis_name="s")

    scratch = (
        [pltpu.VMEM((chunk,), jnp.int32) for _ in range(_NBUF)]
        + [pltpu.VMEM((chunk, dim), jnp.float32) for _ in range(_NBUF)]
        + [pltpu.SemaphoreType.DMA for _ in range(3 * _NBUF)]
    )

    @functools.partial(
        pl.kernel,
        mesh=mesh,
        out_type=jax.ShapeDtypeStruct((total_rows, dim), jnp.float32),
        scratch_types=scratch,
        compiler_params=pltpu.CompilerParams(use_tc_tiling_on_sc=False),
    )
    def gather_kernel(idx_hbm, table_hbm, out_hbm, *bufs):
        idx_v = bufs[0:_NBUF]
        rows_v = bufs[_NBUF:2 * _NBUF]
        sem_i = bufs[2 * _NBUF:3 * _NBUF]
        sem_r = bufs[3 * _NBUF:4 * _NBUF]
        sem_o = bufs[4 * _NBUF:5 * _NBUF]

        wid = lax.axis_index("s") * _NC + lax.axis_index("c")
        base = wid * b_per_w

        for b in range(_NBUF):
            pltpu.async_copy(
                idx_hbm.at[pl.ds(base + b * chunk, chunk)], idx_v[b], sem_i[b]
            )

        def group(g, carry):
            for b in range(_NBUF):
                i = g * _NBUF + b
                off = base + i * chunk

                pltpu.make_async_copy(
                    idx_hbm.at[pl.ds(base, chunk)], idx_v[b], sem_i[b]
                ).wait()

                @pl.when(g > 0)
                def _():
                    pltpu.make_async_copy(
                        rows_v[b], out_hbm.at[pl.ds(base, chunk)], sem_o[b]
                    ).wait()

                pltpu.async_copy(table_hbm.at[idx_v[b]], rows_v[b], sem_r[b]).wait()

                pltpu.async_copy(rows_v[b], out_hbm.at[pl.ds(off, chunk)], sem_o[b])

                @pl.when(i + _NBUF < nchunks)
                def _():
                    pltpu.async_copy(
                        idx_hbm.at[pl.ds(off + _NBUF * chunk, chunk)],
                        idx_v[b],
                        sem_i[b],
                    )

            return carry

        lax.fori_loop(0, nchunks // _NBUF, group, 0)

        for b in range(_NBUF):
            pltpu.make_async_copy(
                rows_v[b], out_hbm.at[pl.ds(base, chunk)], sem_o[b]
            ).wait()

    return gather_kernel


def kernel(token_ids, weights):
    b, h = token_ids.shape
    idx = token_ids.reshape(-1).astype(jnp.int32)
    out = _make_gather(b * h, EMBEDDING_DIM, 1600)(idx, weights)
    return out.reshape(b, h, EMBEDDING_DIM)


# speedup vs baseline: 2.5131x; 2.2620x over previous
import functools

import jax
import jax.numpy as jnp
from jax import lax
from jax.experimental import pallas as pl
from jax.experimental.pallas import tpu as pltpu
from jax.experimental.pallas import tpu_sc as plsc

EMBEDDING_DIM = 32
_NC = 2
_NS = 16
_NW = _NC * _NS
_NBUF = 2


@functools.lru_cache(maxsize=None)
def _make_gather(batch: int, hist: int, dim: int, rows_per_chunk: int):
    b_per_w = batch // _NW
    nchunks = b_per_w // rows_per_chunk
    chunk = rows_per_chunk * hist
    assert nchunks % _NBUF == 0
    hist_pad = (hist + 7) // 8 * 8
    dim_pad = 128
    mesh = plsc.VectorSubcoreMesh(core_axis_name="c", subcore_axis_name="s")

    scratch = (
        [pltpu.VMEM((chunk,), jnp.int32) for _ in range(_NBUF)]
        + [pltpu.VMEM((chunk, dim), jnp.float32) for _ in range(_NBUF)]
        + [pltpu.SemaphoreType.DMA for _ in range(3 * _NBUF)]
    )

    @functools.partial(
        pl.kernel,
        mesh=mesh,
        out_type=jax.ShapeDtypeStruct((batch, hist_pad, dim_pad), jnp.float32),
        scratch_types=scratch,
        compiler_params=pltpu.CompilerParams(use_tc_tiling_on_sc=False),
    )
    def gather_kernel(idx_hbm, table_hbm, out_hbm, *bufs):
        idx_v = bufs[0:_NBUF]
        rows_v = bufs[_NBUF:2 * _NBUF]
        sem_i = bufs[2 * _NBUF:3 * _NBUF]
        sem_r = bufs[3 * _NBUF:4 * _NBUF]
        sem_o = bufs[4 * _NBUF:5 * _NBUF]

        wid = lax.axis_index("s") * _NC + lax.axis_index("c")
        row_base = wid * b_per_w

        def store_wait(b):
            for _ in range(rows_per_chunk):
                pltpu.make_async_copy(
                    rows_v[b].at[pl.ds(0, hist), :],
                    out_hbm.at[0, pl.ds(0, hist), pl.ds(0, dim)],
                    sem_o[b],
                ).wait()

        for b in range(_NBUF):
            pltpu.async_copy(
                idx_hbm.at[pl.ds((row_base + b * rows_per_chunk) * hist, chunk)],
                idx_v[b],
                sem_i[b],
            )

        def group(g, carry):
            for b in range(_NBUF):
                i = g * _NBUF + b
                brow = row_base + i * rows_per_chunk

                pltpu.make_async_copy(
                    idx_hbm.at[pl.ds(0, chunk)], idx_v[b], sem_i[b]
                ).wait()

                @pl.when(g > 0)
                def _():
                    store_wait(b)

                pltpu.async_copy(table_hbm.at[idx_v[b]], rows_v[b], sem_r[b]).wait()

                for r in range(rows_per_chunk):
                    pltpu.async_copy(
                        rows_v[b].at[pl.ds(r * hist, hist), :],
                        out_hbm.at[brow + r, pl.ds(0, hist), pl.ds(0, dim)],
                        sem_o[b],
                    )

                @pl.when(i + _NBUF < nchunks)
                def _():
                    pltpu.async_copy(
                        idx_hbm.at[
                            pl.ds((brow + _NBUF * rows_per_chunk) * hist, chunk)
                        ],
                        idx_v[b],
                        sem_i[b],
                    )

            return carry

        lax.fori_loop(0, nchunks // _NBUF, group, 0)

        for b in range(_NBUF):
            store_wait(b)

    return gather_kernel


def kernel(token_ids, weights):
    b, h = token_ids.shape
    idx = token_ids.reshape(-1).astype(jnp.int32)
    big = _make_gather(b, h, EMBEDDING_DIM, 32)(idx, weights)
    return big[:, :h, :EMBEDDING_DIM]


# speedup vs baseline: 2.5145x; 1.0006x over previous
import functools

import jax
import jax.numpy as jnp
from jax import lax
from jax.experimental import pallas as pl
from jax.experimental.pallas import tpu as pltpu
from jax.experimental.pallas import tpu_sc as plsc

EMBEDDING_DIM = 32
_NC = 2
_NS = 16
_NW = _NC * _NS
_NBUF = 2


@functools.lru_cache(maxsize=None)
def _make_gather(batch: int, hist: int, dim: int, rows_per_chunk: int):
    b_per_w = batch // _NW
    nchunks = b_per_w // rows_per_chunk
    chunk = rows_per_chunk * hist
    assert nchunks % _NBUF == 0
    hist_pad = (hist + 7) // 8 * 8
    dim_pad = 128
    mesh = plsc.VectorSubcoreMesh(core_axis_name="c", subcore_axis_name="s")

    scratch = (
        [pltpu.VMEM((chunk,), jnp.int32) for _ in range(_NBUF)]
        + [pltpu.VMEM((chunk, dim), jnp.float32) for _ in range(_NBUF)]
        + [pltpu.SemaphoreType.DMA for _ in range(3 * _NBUF)]
    )

    @functools.partial(
        pl.kernel,
        mesh=mesh,
        out_type=jax.ShapeDtypeStruct((batch, hist_pad, dim_pad), jnp.float32),
        scratch_types=scratch,
        compiler_params=pltpu.CompilerParams(use_tc_tiling_on_sc=False),
    )
    def gather_kernel(ids_hbm, table_hbm, out_hbm, *bufs):
        idx_v = bufs[0:_NBUF]
        rows_v = bufs[_NBUF:2 * _NBUF]
        sem_i = bufs[2 * _NBUF:3 * _NBUF]
        sem_r = bufs[3 * _NBUF:4 * _NBUF]
        sem_o = bufs[4 * _NBUF:5 * _NBUF]

        wid = lax.axis_index("s") * _NC + lax.axis_index("c")
        row_base = wid * b_per_w

        def ids_start(i, b):
            off = (row_base + i * rows_per_chunk) * hist
            pltpu.async_copy(ids_hbm.at[pl.ds(off, chunk)], idx_v[b], sem_i[b])

        def ids_wait(b):
            pltpu.make_async_copy(
                ids_hbm.at[pl.ds(0, chunk)], idx_v[b], sem_i[b]
            ).wait()

        def store_wait(b):
            for _ in range(rows_per_chunk):
                pltpu.make_async_copy(
                    rows_v[b].at[pl.ds(0, hist), :],
                    out_hbm.at[0, pl.ds(0, hist), pl.ds(0, dim)],
                    sem_o[b],
                ).wait()

        for b in range(_NBUF):
            ids_start(b, b)

        def group(g, carry):
            for b in range(_NBUF):
                i = g * _NBUF + b
                brow = row_base + i * rows_per_chunk

                ids_wait(b)

                @pl.when(g > 0)
                def _():
                    store_wait(b)

                pltpu.async_copy(table_hbm.at[idx_v[b]], rows_v[b], sem_r[b]).wait()

                for r in range(rows_per_chunk):
                    pltpu.async_copy(
                        rows_v[b].at[pl.ds(r * hist, hist), :],
                        out_hbm.at[brow + r, pl.ds(0, hist), pl.ds(0, dim)],
                        sem_o[b],
                    )

                @pl.when(i + _NBUF < nchunks)
                def _():
                    ids_start(i + _NBUF, b)

            return carry

        lax.fori_loop(0, nchunks // _NBUF, group, 0)

        for b in range(_NBUF):
            store_wait(b)

    return gather_kernel


def kernel(token_ids, weights):
    b, h = token_ids.shape
    ids_flat = token_ids.astype(jnp.int32).reshape(b * h)
    big = _make_gather(b, h, EMBEDDING_DIM, 32)(ids_flat, weights)
    return big[:, :h, :EMBEDDING_DIM]
